# baseline (speedup 1.0000x reference)
import functools

import jax
import jax.numpy as jnp
from jax import lax
from jax.experimental import pallas as pl
from jax.experimental.pallas import tpu as pltpu
from jax.experimental.pallas import tpu_sc as plsc

NUM_FIELDS = 26
FIELD_SIZE = 40000
BATCH = 16384
EMBED_DIM = 16
LANES = 16
ITEMS_PER_W = NUM_FIELDS // 2
N_STAGED = 8
SEG_TILES = 313
TAB_TILE_ROWS = 8125
IDX_TOTAL = NUM_FIELDS * BATCH
IDX_CHUNK = IDX_TOTAL // 16


def _sc_lookup(table_gv, idx_gv, idx_tail):
    mesh = plsc.VectorSubcoreMesh(core_axis_name="c", subcore_axis_name="s")

    @functools.partial(
        pl.kernel,
        out_type=jax.ShapeDtypeStruct((NUM_FIELDS, 256, 1024), jnp.float32),
        mesh=mesh,
        scratch_types=[
            pltpu.VMEM((SEG_TILES, 128), jnp.float32),
            pltpu.VMEM((SEG_TILES, 128), jnp.float32),
            pltpu.VMEM((128, 128), jnp.int32),
            pltpu.VMEM((128, 128), jnp.int32),
            pltpu.VMEM((64, 128), jnp.float32),
            pltpu.VMEM((64, 128), jnp.float32),
            pltpu.SemaphoreType.DMA,
            pltpu.SemaphoreType.DMA,
            pltpu.SemaphoreType.DMA,
            pltpu.SemaphoreType.DMA,
            pltpu.SemaphoreType.DMA,
            pltpu.SemaphoreType.DMA,
        ],
        compiler_params=pltpu.CompilerParams(
            use_tc_tiling_on_sc=False, needs_layout_passes=False
        ),
    )
    def k(table_hbm, idx_hbm, tail_hbm, out_hbm, seg_v0, seg_v1, idx_v0,
          idx_v1, out_v0, out_v1, sseg0, sseg1, sidx0, sidx1, sout0, sout1):
        c = lax.axis_index("c")
        e = lax.axis_index("s")
        t = e // 8
        s = e % 8
        shift = 64 * c
        segs, idxs = [seg_v0, seg_v1], [idx_v0, idx_v1]
        outs = [out_v0, out_v1]
        ssegs, sidxs, souts = [sseg0, sseg1], [sidx0, sidx1], [sout0, sout1]

        def seg_src(f):
            j0 = (f * FIELD_SIZE - shift) // 128
            return table_hbm.at[pl.ds(t * TAB_TILE_ROWS + j0, SEG_TILES),
                                pl.ds(128 * s, 128)]

        seg_desc = [
            pltpu.async_copy(seg_src(c), segs[0], ssegs[0]),
            pltpu.async_copy(seg_src(c + 2), segs[1], ssegs[1]),
        ]

        def idx_src(i):
            f = c + 2 * i
            if i < 12:
                fr = pl.multiple_of((f // 8) * 128, 128)
                fc = pl.multiple_of((f % 8) * 128, 128)
                return idx_hbm.at[pl.ds(fr, 128), pl.ds(fc, 128)]
            return tail_hbm.at[c]

        idx_desc = [
            pltpu.async_copy(idx_src(0), idxs[0], sidxs[0]),
            pltpu.async_copy(idx_src(1), idxs[1], sidxs[1]),
        ]

        out_desc = [None, None]
        for i in range(ITEMS_PER_W):
            f = c + 2 * i
            b = i % 2
            seg_desc[b].wait()
            idx_desc[b].wait()
            seg_b, idx_b = segs[b], idxs[b]

            for h in range(2):
                if out_desc[h] is not None:
                    out_desc[h].wait()
                out_h = outs[h]

                @plsc.parallel_loop(0, 64, unroll=2)
                def gather_body(r):
                    for u in range(8):
                        li = idx_b[64 * h + r, pl.ds(16 * u, LANES)]
                        li = li + shift
                        vals = plsc.load_gather(
                            seg_b, [lax.shift_right_logical(li, 7),
                                    lax.bitwise_and(li, 127)]
                        )
                        out_h[r, pl.ds(16 * u, LANES)] = vals

                out_desc[h] = pltpu.async_copy(
                    out_h,
                    out_hbm.at[f, pl.ds(t * 128 + 64 * h, 64), pl.ds(128 * s, 128)],
                    souts[h],
                )
            if i + 2 < ITEMS_PER_W:
                seg_desc[b] = pltpu.async_copy(
                    seg_src(f + 4), segs[b], ssegs[b]
                )
                idx_desc[b] = pltpu.async_copy(idx_src(i + 2), idxs[b], sidxs[b])
        out_desc[0].wait()
        out_desc[1].wait()

    return k(table_gv, idx_gv, idx_tail)


def kernel(inputs, table):
    table_gv = (
        table.T.reshape(2, 8, TAB_TILE_ROWS, 128)
        .transpose(0, 2, 1, 3)
        .reshape(2 * TAB_TILE_ROWS, 1024)
    )
    inputs_t = inputs.astype(jnp.int32).T
    idx_gv = (
        inputs_t[:24]
        .reshape(3, 8, 128, 128)
        .transpose(0, 2, 1, 3)
        .reshape(384, 1024)
    )
    idx_tail = inputs_t[24:].reshape(2, 128, 128)
    out_gv = _sc_lookup(table_gv, idx_gv, idx_tail)
    return (
        out_gv.reshape(NUM_FIELDS, 2, 128, 8, 128)
        .transpose(2, 4, 0, 1, 3)
        .reshape(BATCH, NUM_FIELDS, EMBED_DIM)
    )


# speedup vs baseline: 1.0110x; 1.0110x over previous
import functools

import jax
import jax.numpy as jnp
from jax import lax
from jax.experimental import pallas as pl
from jax.experimental.pallas import tpu as pltpu
from jax.experimental.pallas import tpu_sc as plsc

NUM_FIELDS = 26
FIELD_SIZE = 40000
BATCH = 16384
EMBED_DIM = 16
LANES = 16
ITEMS_PER_W = NUM_FIELDS // 2
N_STAGED = 8
SEG_TILES = 313
TAB_TILE_ROWS = 8125
IDX_TOTAL = NUM_FIELDS * BATCH
IDX_CHUNK = IDX_TOTAL // 16


def _sc_lookup(table_gv, idx_flat):
    mesh = plsc.VectorSubcoreMesh(core_axis_name="c", subcore_axis_name="s")

    @functools.partial(
        pl.kernel,
        out_type=jax.ShapeDtypeStruct((NUM_FIELDS, 256, 1024), jnp.float32),
        mesh=mesh,
        scratch_types=[
            pltpu.VMEM((SEG_TILES, 128), jnp.float32),
            pltpu.VMEM((SEG_TILES, 128), jnp.float32),
            pltpu.VMEM((BATCH,), jnp.int32),
            pltpu.VMEM((BATCH,), jnp.int32),
            pltpu.VMEM((64, 128), jnp.float32),
            pltpu.VMEM((64, 128), jnp.float32),
            pltpu.SemaphoreType.DMA,
            pltpu.SemaphoreType.DMA,
            pltpu.SemaphoreType.DMA,
            pltpu.SemaphoreType.DMA,
            pltpu.SemaphoreType.DMA,
            pltpu.SemaphoreType.DMA,
        ],
        compiler_params=pltpu.CompilerParams(
            use_tc_tiling_on_sc=False, needs_layout_passes=False
        ),
    )
    def k(table_hbm, idx_hbm, out_hbm, seg_v0, seg_v1, idx_v0, idx_v1,
          out_v0, out_v1, sseg0, sseg1, sidx0, sidx1, sout0, sout1):
        c = lax.axis_index("c")
        e = lax.axis_index("s")
        t = e // 8
        s = e % 8
        shift = 64 * c
        segs, idxs = [seg_v0, seg_v1], [idx_v0, idx_v1]
        outs = [out_v0, out_v1]
        ssegs, sidxs, souts = [sseg0, sseg1], [sidx0, sidx1], [sout0, sout1]

        def seg_src(f):
            j0 = (f * FIELD_SIZE - shift) // 128
            return table_hbm.at[pl.ds(t * TAB_TILE_ROWS + j0, SEG_TILES),
                                pl.ds(128 * s, 128)]

        seg_desc = [
            pltpu.async_copy(seg_src(c), segs[0], ssegs[0]),
            pltpu.async_copy(seg_src(c + 2), segs[1], ssegs[1]),
        ]

        def idx_src(i):
            return idx_hbm.at[pl.ds((c + 2 * i) * BATCH, BATCH)]

        idx_desc = [
            pltpu.async_copy(idx_src(0), idxs[0], sidxs[0]),
            pltpu.async_copy(idx_src(1), idxs[1], sidxs[1]),
        ]

        out_desc = [None, None]
        for i in range(ITEMS_PER_W):
            f = c + 2 * i
            b = i % 2
            seg_desc[b].wait()
            idx_desc[b].wait()
            seg_b, idx_b = segs[b], idxs[b]

            for h in range(2):
                if out_desc[h] is not None:
                    out_desc[h].wait()
                out_h = outs[h]

                @plsc.parallel_loop(0, 64, unroll=2)
                def gather_body(r):
                    for u in range(8):
                        li = idx_b[pl.ds(((64 * h + r) * 8 + u) * LANES, LANES)]
                        li = li + shift
                        vals = plsc.load_gather(
                            seg_b, [lax.shift_right_logical(li, 7),
                                    lax.bitwise_and(li, 127)]
                        )
                        out_h[r, pl.ds(16 * u, LANES)] = vals

                out_desc[h] = pltpu.async_copy(
                    out_h,
                    out_hbm.at[f, pl.ds(t * 128 + 64 * h, 64), pl.ds(128 * s, 128)],
                    souts[h],
                )
            if i + 2 < ITEMS_PER_W:
                seg_desc[b] = pltpu.async_copy(
                    seg_src(f + 4), segs[b], ssegs[b]
                )
                idx_desc[b] = pltpu.async_copy(idx_src(i + 2), idxs[b], sidxs[b])
        out_desc[0].wait()
        out_desc[1].wait()

    return k(table_gv, idx_flat)


def kernel(inputs, table):
    table_gv = (
        table.T.reshape(2, 8, TAB_TILE_ROWS, 128)
        .transpose(0, 2, 1, 3)
        .reshape(2 * TAB_TILE_ROWS, 1024)
    )
    idx_flat = inputs.astype(jnp.int32).T.reshape(IDX_TOTAL)
    out_gv = _sc_lookup(table_gv, idx_flat)
    return (
        out_gv.reshape(NUM_FIELDS, 2, 128, 8, 128)
        .transpose(2, 4, 0, 1, 3)
        .reshape(BATCH, NUM_FIELDS, EMBED_DIM)
    )


# speedup vs baseline: 1.0137x; 1.0027x over previous
import functools

import jax
import jax.numpy as jnp
from jax import lax
from jax.experimental import pallas as pl
from jax.experimental.pallas import tpu as pltpu
from jax.experimental.pallas import tpu_sc as plsc

NUM_FIELDS = 26
FIELD_SIZE = 40000
BATCH = 16384
EMBED_DIM = 16
LANES = 16
ITEMS_PER_W = NUM_FIELDS // 2
N_STAGED = 8
SEG_TILES = 313
TAB_TILE_ROWS = 8125
IDX_TOTAL = NUM_FIELDS * BATCH
IDX_CHUNK = IDX_TOTAL // 16


def _sc_lookup(table_gv, idx_flat):
    mesh = plsc.VectorSubcoreMesh(core_axis_name="c", subcore_axis_name="s")

    @functools.partial(
        pl.kernel,
        out_type=jax.ShapeDtypeStruct((NUM_FIELDS, 256, 1024), jnp.float32),
        mesh=mesh,
        scratch_types=[
            pltpu.VMEM((SEG_TILES, 128), jnp.float32),
            pltpu.VMEM((SEG_TILES, 128), jnp.float32),
            pltpu.VMEM((BATCH,), jnp.int32),
            pltpu.VMEM((BATCH,), jnp.int32),
            pltpu.VMEM((64, 128), jnp.float32),
            pltpu.VMEM((64, 128), jnp.float32),
            pltpu.SemaphoreType.DMA,
            pltpu.SemaphoreType.DMA,
            pltpu.SemaphoreType.DMA,
            pltpu.SemaphoreType.DMA,
            pltpu.SemaphoreType.DMA,
            pltpu.SemaphoreType.DMA,
        ],
        compiler_params=pltpu.CompilerParams(
            use_tc_tiling_on_sc=False, needs_layout_passes=False,
            skip_device_barrier=True
        ),
    )
    def k(table_hbm, idx_hbm, out_hbm, seg_v0, seg_v1, idx_v0, idx_v1,
          out_v0, out_v1, sseg0, sseg1, sidx0, sidx1, sout0, sout1):
        c = lax.axis_index("c")
        e = lax.axis_index("s")
        t = e // 8
        s = e % 8
        shift = 64 * c
        segs, idxs = [seg_v0, seg_v1], [idx_v0, idx_v1]
        outs = [out_v0, out_v1]
        ssegs, sidxs, souts = [sseg0, sseg1], [sidx0, sidx1], [sout0, sout1]

        def seg_src(f):
            j0 = (f * FIELD_SIZE - shift) // 128
            return table_hbm.at[pl.ds(t * TAB_TILE_ROWS + j0, SEG_TILES),
                                pl.ds(128 * s, 128)]

        seg_desc = [
            pltpu.async_copy(seg_src(c), segs[0], ssegs[0]),
            pltpu.async_copy(seg_src(c + 2), segs[1], ssegs[1]),
        ]

        def idx_src(i):
            return idx_hbm.at[pl.ds((c + 2 * i) * BATCH, BATCH)]

        idx_desc = [
            pltpu.async_copy(idx_src(0), idxs[0], sidxs[0]),
            pltpu.async_copy(idx_src(1), idxs[1], sidxs[1]),
        ]

        out_desc = [None, None]
        for i in range(ITEMS_PER_W):
            f = c + 2 * i
            b = i % 2
            seg_desc[b].wait()
            idx_desc[b].wait()
            seg_b, idx_b = segs[b], idxs[b]

            for h in range(2):
                if out_desc[h] is not None:
                    out_desc[h].wait()
                out_h = outs[h]

                @plsc.parallel_loop(0, 64, unroll=2)
                def gather_body(r):
                    for u in range(8):
                        li = idx_b[pl.ds(((64 * h + r) * 8 + u) * LANES, LANES)]
                        li = li + shift
                        vals = plsc.load_gather(
                            seg_b, [lax.shift_right_logical(li, 7),
                                    lax.bitwise_and(li, 127)]
                        )
                        out_h[r, pl.ds(16 * u, LANES)] = vals

                out_desc[h] = pltpu.async_copy(
                    out_h,
                    out_hbm.at[f, pl.ds(t * 128 + 64 * h, 64), pl.ds(128 * s, 128)],
                    souts[h],
                )
            if i + 2 < ITEMS_PER_W:
                seg_desc[b] = pltpu.async_copy(
                    seg_src(f + 4), segs[b], ssegs[b]
                )
                idx_desc[b] = pltpu.async_copy(idx_src(i + 2), idxs[b], sidxs[b])
        out_desc[0].wait()
        out_desc[1].wait()

    return k(table_gv, idx_flat)


def kernel(inputs, table):
    table_gv = (
        table.T.reshape(2, 8, TAB_TILE_ROWS, 128)
        .transpose(0, 2, 1, 3)
        .reshape(2 * TAB_TILE_ROWS, 1024)
    )
    idx_flat = inputs.astype(jnp.int32).T.reshape(IDX_TOTAL)
    out_gv = _sc_lookup(table_gv, idx_flat)
    return (
        out_gv.reshape(NUM_FIELDS, 2, 128, 8, 128)
        .transpose(2, 4, 0, 1, 3)
        .reshape(BATCH, NUM_FIELDS, EMBED_DIM)
    )
